# baseline (device time: 107186 ns/iter reference)
import jax
import jax.numpy as jnp
from jax import lax
from jax.experimental import pallas as pl
from jax.experimental.pallas import tpu as pltpu

N_DEV = 16
N_HOPS = 8
SUBS = 4


def _perm_of(t):
    c = t // 4
    u = t % 4
    z = jnp.where(c % 2 == 0, u, 3 - u)
    return z * 4 + c


def kernel(x, w_mat, scale_x, scale_w):
    m_per, k = x.shape
    _, n_per = w_mat.shape
    m_total = N_DEV * m_per
    rows = m_per // SUBS

    my = lax.axis_index("i")
    mz = my // 4
    mc = my % 4
    r = 4 * mc + jnp.where(mc % 2 == 0, mz, 3 - mz)
    nbrs = jnp.stack([_perm_of((r + N_DEV - 1) % N_DEV),
                      _perm_of((r + 1) % N_DEV)])
    hs = jnp.arange(N_HOPS, dtype=jnp.int32)
    cw_org = _perm_of((r + N_DEV - 1 - hs) % N_DEV)
    ccw_org = _perm_of((r + 1 + hs) % N_DEV)

    def cw_has(h, s):
        return h < N_HOPS - 1 or s < SUBS // 2

    def ccw_has(h, s):
        return h < N_HOPS - 1 or s >= SUBS // 2

    def body(x_ref, w_ref, sx_ref, sw_ref, nbr_ref, cw_org_ref, ccw_org_ref,
             out_ref, x8_ref, w8_ref, cw_ref, ccw_ref,
             cw_send, cw_recv, ccw_send, ccw_recv):
        left = nbr_ref[0]
        right = nbr_ref[1]

        x8_ref[...] = x_ref[...].astype(jnp.float8_e4m3fn)

        barrier_sem = pltpu.get_barrier_semaphore()
        for nbr in (left, right):
            pl.semaphore_signal(
                barrier_sem, inc=1,
                device_id=(nbr,), device_id_type=pl.DeviceIdType.MESH,
            )
        pl.semaphore_wait(barrier_sem, 2)

        scale = sx_ref[0] * sw_ref[0]

        def gemm_store(chunk, row_start):
            acc = lax.dot_general(
                chunk, w8_ref[...],
                (((1,), (0,)), ((), ())),
                preferred_element_type=jnp.float32,
            )
            y = acc * scale
            z = y / (1.0 + jnp.exp(-jnp.clip(y, -60.0, 60.0)))
            out_ref[pl.ds(row_start, chunk.shape[0]), :] = z

        def mk(src, dst_buf, h, s, send_sems, recv_sems, dev):
            return pltpu.make_async_remote_copy(
                src_ref=src,
                dst_ref=dst_buf.at[h, s],
                send_sem=send_sems.at[h, s],
                recv_sem=recv_sems.at[h, s],
                device_id=(dev,),
                device_id_type=pl.DeviceIdType.MESH,
            )

        sends = []

        for s in range(SUBS):
            piece = x8_ref.at[pl.ds(s * rows, rows)]
            rd = mk(piece, cw_ref, 0, s, cw_send, cw_recv, right)
            rd.start()
            sends.append(rd)
            rd = mk(piece, ccw_ref, 0, s, ccw_send, ccw_recv, left)
            rd.start()
            sends.append(rd)

        w8_ref[...] = w_ref[...].astype(jnp.float8_e4m3fn)
        gemm_store(x8_ref[...], lax.axis_index("i") * m_per)

        def recv_cw(h, s):
            return mk(x8_ref.at[pl.ds(0, rows)], cw_ref, h, s,
                      cw_send, cw_recv, right)

        def recv_ccw(h, s):
            return mk(x8_ref.at[pl.ds(0, rows)], ccw_ref, h, s,
                      ccw_send, ccw_recv, left)

        for h in range(N_HOPS):
            for s in range(SUBS):
                if cw_has(h, s):
                    recv_cw(h, s).wait_recv()
                    if h + 1 < N_HOPS and cw_has(h + 1, s):
                        rd = mk(cw_ref.at[h, s], cw_ref, h + 1, s,
                                cw_send, cw_recv, right)
                        rd.start()
                        sends.append(rd)
                if ccw_has(h, s):
                    recv_ccw(h, s).wait_recv()
                    if h + 1 < N_HOPS and ccw_has(h + 1, s):
                        rd = mk(ccw_ref.at[h, s], ccw_ref, h + 1, s,
                                ccw_send, ccw_recv, left)
                        rd.start()
                        sends.append(rd)
            for s in range(SUBS):
                if cw_has(h, s):
                    gemm_store(cw_ref[h, s], cw_org_ref[h] * m_per + s * rows)
                if ccw_has(h, s):
                    gemm_store(ccw_ref[h, s], ccw_org_ref[h] * m_per + s * rows)

        for rd in sends:
            rd.wait_send()

    f8 = jnp.float8_e4m3fn
    return pl.pallas_call(
        body,
        out_shape=jax.ShapeDtypeStruct((m_total, n_per), jnp.float32),
        in_specs=[
            pl.BlockSpec(memory_space=pltpu.VMEM),
            pl.BlockSpec(memory_space=pltpu.VMEM),
            pl.BlockSpec(memory_space=pltpu.SMEM),
            pl.BlockSpec(memory_space=pltpu.SMEM),
            pl.BlockSpec(memory_space=pltpu.SMEM),
            pl.BlockSpec(memory_space=pltpu.SMEM),
            pl.BlockSpec(memory_space=pltpu.SMEM),
        ],
        out_specs=pl.BlockSpec(memory_space=pltpu.VMEM),
        scratch_shapes=[
            pltpu.VMEM((m_per, k), f8),
            pltpu.VMEM((k, n_per), f8),
            pltpu.VMEM((N_HOPS, SUBS, rows, k), f8),
            pltpu.VMEM((N_HOPS, SUBS, rows, k), f8),
            pltpu.SemaphoreType.DMA((N_HOPS, SUBS)),
            pltpu.SemaphoreType.DMA((N_HOPS, SUBS)),
            pltpu.SemaphoreType.DMA((N_HOPS, SUBS)),
            pltpu.SemaphoreType.DMA((N_HOPS, SUBS)),
        ],
        compiler_params=pltpu.CompilerParams(
            collective_id=0,
            vmem_limit_bytes=44 * 1024 * 1024,
        ),
    )(x, w_mat, scale_x, scale_w, nbrs, cw_org, ccw_org)


# device time: 101438 ns/iter; 1.0567x vs baseline; 1.0567x over previous
import jax
import jax.numpy as jnp
from jax import lax
from jax.experimental import pallas as pl
from jax.experimental.pallas import tpu as pltpu

N_DEV = 16
N_HOPS = 8
SUBS = 4


def _perm_of(t):
    c = t // 4
    u = t % 4
    z = jnp.where(c % 2 == 0, u, 3 - u)
    return z * 4 + c


def kernel(x, w_mat, scale_x, scale_w):
    m_per, k = x.shape
    _, n_per = w_mat.shape
    m_total = N_DEV * m_per
    rows = m_per // SUBS

    x = x.astype(jnp.float8_e4m3fn)
    w_mat = w_mat.astype(jnp.float8_e4m3fn)

    my = lax.axis_index("i")
    mz = my // 4
    mc = my % 4
    r = 4 * mc + jnp.where(mc % 2 == 0, mz, 3 - mz)
    nbrs = jnp.stack([_perm_of((r + N_DEV - 1) % N_DEV),
                      _perm_of((r + 1) % N_DEV)])
    hs = jnp.arange(N_HOPS, dtype=jnp.int32)
    cw_org = _perm_of((r + N_DEV - 1 - hs) % N_DEV)
    ccw_org = _perm_of((r + 1 + hs) % N_DEV)

    def cw_has(h, s):
        return h < N_HOPS - 1 or s < SUBS // 2

    def ccw_has(h, s):
        return h < N_HOPS - 1 or s >= SUBS // 2

    def body(x_ref, w_ref, sx_ref, sw_ref, nbr_ref, cw_org_ref, ccw_org_ref,
             out_ref, stage_ref, cw_ref, ccw_ref,
             cw_send, cw_recv, ccw_send, ccw_recv, out_sems):
        left = nbr_ref[0]
        right = nbr_ref[1]

        barrier_sem = pltpu.get_barrier_semaphore()
        for nbr in (left, right):
            pl.semaphore_signal(
                barrier_sem, inc=1,
                device_id=(nbr,), device_id_type=pl.DeviceIdType.MESH,
            )
        pl.semaphore_wait(barrier_sem, 2)

        scale = sx_ref[0] * sw_ref[0]

        def gemm_store(chunk, row_start):
            acc = lax.dot_general(
                chunk, w_ref[...],
                (((1,), (0,)), ((), ())),
                preferred_element_type=jnp.float32,
            )
            y = acc * scale
            z = y / (1.0 + jnp.exp(-jnp.clip(y, -60.0, 60.0)))
            stage_ref[pl.ds(row_start, chunk.shape[0]), :] = z

        out_copies = []

        def flush(row_start, nrows, idx):
            cp = pltpu.make_async_copy(
                stage_ref.at[pl.ds(row_start, nrows)],
                out_ref.at[pl.ds(row_start, nrows)],
                out_sems.at[idx],
            )
            cp.start()
            out_copies.append(cp)

        def mk(src, dst_buf, h, s, send_sems, recv_sems, dev):
            return pltpu.make_async_remote_copy(
                src_ref=src,
                dst_ref=dst_buf.at[h, s],
                send_sem=send_sems.at[h, s],
                recv_sem=recv_sems.at[h, s],
                device_id=(dev,),
                device_id_type=pl.DeviceIdType.MESH,
            )

        sends = []

        for s in range(SUBS):
            piece = x_ref.at[pl.ds(s * rows, rows)]
            rd = mk(piece, cw_ref, 0, s, cw_send, cw_recv, right)
            rd.start()
            sends.append(rd)
            rd = mk(piece, ccw_ref, 0, s, ccw_send, ccw_recv, left)
            rd.start()
            sends.append(rd)
        gemm_store(x_ref[...], lax.axis_index("i") * m_per)
        flush(lax.axis_index("i") * m_per, m_per, 0)

        def recv_cw(h, s):
            return mk(x_ref.at[pl.ds(0, rows)], cw_ref, h, s,
                      cw_send, cw_recv, right)

        def recv_ccw(h, s):
            return mk(x_ref.at[pl.ds(0, rows)], ccw_ref, h, s,
                      ccw_send, ccw_recv, left)

        for h in range(N_HOPS):
            for s in range(SUBS):
                if cw_has(h, s):
                    recv_cw(h, s).wait_recv()
                    if h + 1 < N_HOPS and cw_has(h + 1, s):
                        rd = mk(cw_ref.at[h, s], cw_ref, h + 1, s,
                                cw_send, cw_recv, right)
                        rd.start()
                        sends.append(rd)
                if ccw_has(h, s):
                    recv_ccw(h, s).wait_recv()
                    if h + 1 < N_HOPS and ccw_has(h + 1, s):
                        rd = mk(ccw_ref.at[h, s], ccw_ref, h + 1, s,
                                ccw_send, ccw_recv, left)
                        rd.start()
                        sends.append(rd)
            for s in range(SUBS):
                if cw_has(h, s):
                    gemm_store(cw_ref[h, s], cw_org_ref[h] * m_per + s * rows)
                if ccw_has(h, s):
                    gemm_store(ccw_ref[h, s], ccw_org_ref[h] * m_per + s * rows)
            if h < N_HOPS - 1:
                flush(cw_org_ref[h] * m_per, m_per, 1 + h)
                flush(ccw_org_ref[h] * m_per, m_per, 1 + N_HOPS + h)
            else:
                half = (SUBS // 2) * rows
                flush(cw_org_ref[h] * m_per, half, 1 + h)
                flush(ccw_org_ref[h] * m_per + half, half, 1 + N_HOPS + h)

        for rd in sends:
            rd.wait_send()
        for cp in out_copies:
            cp.wait()

    f8 = jnp.float8_e4m3fn
    return pl.pallas_call(
        body,
        out_shape=jax.ShapeDtypeStruct((m_total, n_per), jnp.float32),
        in_specs=[
            pl.BlockSpec(memory_space=pltpu.VMEM),
            pl.BlockSpec(memory_space=pltpu.VMEM),
            pl.BlockSpec(memory_space=pltpu.SMEM),
            pl.BlockSpec(memory_space=pltpu.SMEM),
            pl.BlockSpec(memory_space=pltpu.SMEM),
            pl.BlockSpec(memory_space=pltpu.SMEM),
            pl.BlockSpec(memory_space=pltpu.SMEM),
        ],
        out_specs=pl.BlockSpec(memory_space=pl.ANY),
        scratch_shapes=[
            pltpu.VMEM((m_total, n_per), jnp.float32),
            pltpu.VMEM((N_HOPS, SUBS, rows, k), f8),
            pltpu.VMEM((N_HOPS, SUBS, rows, k), f8),
            pltpu.SemaphoreType.DMA((N_HOPS, SUBS)),
            pltpu.SemaphoreType.DMA((N_HOPS, SUBS)),
            pltpu.SemaphoreType.DMA((N_HOPS, SUBS)),
            pltpu.SemaphoreType.DMA((N_HOPS, SUBS)),
            pltpu.SemaphoreType.DMA((2 * N_HOPS + 1,)),
        ],
        compiler_params=pltpu.CompilerParams(collective_id=0),
    )(x, w_mat, scale_x, scale_w, nbrs, cw_org, ccw_org)
